# Initial kernel scaffold; baseline (speedup 1.0000x reference)
import jax
import jax.numpy as jnp
from jax.experimental import pallas as pl


def kernel(distances, basis_values):
    raise NotImplementedError("write your pallas kernel here")


# speedup vs baseline: 15.5819x; 15.5819x over previous
import jax
import jax.numpy as jnp
from jax.experimental import pallas as pl
from jax.experimental.pallas import tpu as pltpu

_BLK = 2048


def _body(d_ref, t_ref, o_ref):
    d = d_ref[:, :]
    idxf = jnp.clip(d, 0.0, 1.0 - 1e-6) * 255.0
    k = jax.lax.broadcasted_iota(jnp.int32, (1, 256), 1).astype(jnp.float32)
    w = jnp.maximum(0.0, 1.0 - jnp.abs(idxf - k))
    o_ref[:, :] = jnp.dot(w, t_ref[:, :], preferred_element_type=jnp.float32)


def kernel(distances, basis_values):
    num_basis, num_functions, domain_size = basis_values.shape
    orig_shape = distances.shape
    n = distances.size
    d2 = distances.reshape(n, 1)
    table = basis_values.transpose(2, 0, 1).reshape(domain_size,
                                                    num_basis * num_functions)
    grid = n // _BLK
    out = pl.pallas_call(
        _body,
        grid=(grid,),
        in_specs=[
            pl.BlockSpec((_BLK, 1), lambda i: (i, 0)),
            pl.BlockSpec((domain_size, num_basis * num_functions),
                         lambda i: (0, 0)),
        ],
        out_specs=pl.BlockSpec((_BLK, num_basis * num_functions),
                               lambda i: (i, 0)),
        out_shape=jax.ShapeDtypeStruct((n, num_basis * num_functions),
                                       jnp.float32),
        compiler_params=pltpu.CompilerParams(
            dimension_semantics=("arbitrary",)),
    )(d2, table)
    return out.reshape(*orig_shape, num_basis, num_functions)
